# baseline (device time: 103906 ns/iter reference)
import jax
import jax.numpy as jnp
from jax import lax
from jax.experimental import pallas as pl
from jax.experimental.pallas import tpu as pltpu

M = 2048
K_SHARD = 8192
KB = 4096
N_K = K_SHARD // KB
DYC = 1024
H = M // 2
TW = 512
N_T = M // TW
CW = 256
N_C = M // CW

_MESH = pl.DeviceIdType.MESH


def kernel(dy, W):
    def body(dy_ref, w_ref, out_ref, dyv_ref, dyf_ref, wv_ref,
             rx_ref, acc_ref,
             entry_sems, dy_sems, xs_sems, xr_sems, ys_sems, yr_sems):
        c = pl.program_id(0)
        k = pl.program_id(1)
        my_x = lax.axis_index("x")
        my_y = lax.axis_index("y")
        y_nbr = (my_x, 1 - my_y)
        x_nbr = (1 - my_x, my_y)
        mine_rows = pl.ds(my_y * H, H)

        @pl.when((c == 0) & (k == 0))
        def _():
            pl.semaphore_signal(entry_sems.at[0], inc=1, device_id=x_nbr,
                                device_id_type=_MESH)
            pl.semaphore_signal(entry_sems.at[1], inc=1, device_id=y_nbr,
                                device_id_type=_MESH)
            pl.semaphore_wait(entry_sems.at[0], 1)
            pl.semaphore_wait(entry_sems.at[1], 1)

        def dy_dma(j):
            return pltpu.make_async_copy(
                dy_ref.at[mine_rows, pl.ds(j * DYC, DYC)],
                dyf_ref.at[j % 2],
                dy_sems.at[j % 2],
            )

        @pl.when((c == 0) & (k == 0))
        def _():
            dy_dma(0).start()
            dy_dma(1).start()

        @pl.when(c == 0)
        def _():
            for jo in (0, 1, 2, 3):
                j = 4 * k + jo
                dy_dma(j).wait()

                @pl.when(j + 2 < K_SHARD // DYC)
                def _(j=j):
                    dy_dma(j + 2).start()

                dyv_ref[:, pl.ds(j * DYC, DYC)] = (
                    dyf_ref[j % 2].astype(jnp.bfloat16)
                )

        wv_ref[...] = w_ref[...].astype(jnp.bfloat16)
        part = lax.dot_general(
            dyv_ref[:, pl.ds(k * KB, KB)],
            wv_ref[...],
            (((1,), (1,)), ((), ())),
            preferred_element_type=jnp.float32,
        )

        @pl.when(k == 0)
        def _():
            acc_ref[...] = part

        @pl.when(k == N_K - 1)
        def _():
            out_ref[mine_rows, pl.ds(c * TW, TW)] = (
                (acc_ref[...] + part).astype(jnp.bfloat16)
            )

        def xex(ch):
            cols = pl.ds(ch * CW, CW)
            return pltpu.make_async_remote_copy(
                src_ref=out_ref.at[mine_rows, cols],
                dst_ref=rx_ref.at[ch],
                send_sem=xs_sems.at[ch], recv_sem=xr_sems.at[ch],
                device_id=x_nbr, device_id_type=_MESH,
            )

        def yex(ch):
            cols = pl.ds(ch * CW, CW)
            return pltpu.make_async_remote_copy(
                src_ref=out_ref.at[mine_rows, cols],
                dst_ref=out_ref.at[mine_rows, cols],
                send_sem=ys_sems.at[ch], recv_sem=yr_sems.at[ch],
                device_id=y_nbr, device_id_type=_MESH,
            )

        def finish_x(ch):
            xex(ch).wait()
            cols = pl.ds(ch * CW, CW)
            out_ref[mine_rows, cols] = (
                out_ref[mine_rows, cols].astype(jnp.float32)
                + rx_ref[ch].astype(jnp.float32)
            ).astype(jnp.bfloat16)
            yex(ch).start()

        for cc in range(1, N_T):
            @pl.when((c == cc) & (k == N_K - 1))
            def _(cc=cc):
                finish_x(2 * (cc - 1))
                finish_x(2 * (cc - 1) + 1)

        @pl.when(k == N_K - 1)
        def _():
            xex(2 * c).start()
            xex(2 * c + 1).start()

        @pl.when((c == N_T - 1) & (k == N_K - 1))
        def _():
            finish_x(2 * (N_T - 1))
            finish_x(2 * (N_T - 1) + 1)
            for ch in range(N_C):
                yex(ch).wait()

    return pl.pallas_call(
        body,
        grid=(N_T, N_K),
        out_shape=jax.ShapeDtypeStruct((M, M), jnp.bfloat16),
        in_specs=[
            pl.BlockSpec(memory_space=pltpu.MemorySpace.HBM),
            pl.BlockSpec((TW, KB), lambda c, k: (c, k)),
        ],
        out_specs=pl.BlockSpec((M, M), lambda c, k: (0, 0)),
        scratch_shapes=[
            pltpu.VMEM((H, K_SHARD), jnp.bfloat16),
            pltpu.VMEM((2, H, DYC), jnp.float32),
            pltpu.VMEM((TW, KB), jnp.bfloat16),
            pltpu.VMEM((N_C, H, CW), jnp.bfloat16),
            pltpu.VMEM((H, TW), jnp.float32),
            pltpu.SemaphoreType.REGULAR((2,)),
            pltpu.SemaphoreType.DMA((2,)),
            pltpu.SemaphoreType.DMA((N_C,)),
            pltpu.SemaphoreType.DMA((N_C,)),
            pltpu.SemaphoreType.DMA((N_C,)),
            pltpu.SemaphoreType.DMA((N_C,)),
        ],
        compiler_params=pltpu.CompilerParams(
            vmem_limit_bytes=100 * 1024 * 1024,
        ),
    )(dy, W)


# device time: 99910 ns/iter; 1.0400x vs baseline; 1.0400x over previous
import jax
import jax.numpy as jnp
from jax import lax
from jax.experimental import pallas as pl
from jax.experimental.pallas import tpu as pltpu

M = 2048
K_SHARD = 8192
KB = 2048
N_K = K_SHARD // KB
DYC = 1024
H = M // 2
TW = 512
N_T = M // TW
CW = 256
N_C = M // CW

_MESH = pl.DeviceIdType.MESH


def kernel(dy, W):
    def body(dy_ref, w_ref, out_ref, dyv_ref, dyf_ref, wv_ref,
             rx_ref, acc_ref,
             entry_sems, dy_sems, xs_sems, xr_sems, ys_sems, yr_sems):
        c = pl.program_id(0)
        k = pl.program_id(1)
        my_x = lax.axis_index("x")
        my_y = lax.axis_index("y")
        y_nbr = (my_x, 1 - my_y)
        x_nbr = (1 - my_x, my_y)
        mine_rows = pl.ds(my_y * H, H)

        @pl.when((c == 0) & (k == 0))
        def _():
            pl.semaphore_signal(entry_sems.at[0], inc=1, device_id=x_nbr,
                                device_id_type=_MESH)
            pl.semaphore_signal(entry_sems.at[1], inc=1, device_id=y_nbr,
                                device_id_type=_MESH)
            pl.semaphore_wait(entry_sems.at[0], 1)
            pl.semaphore_wait(entry_sems.at[1], 1)

        def dy_dma(j):
            return pltpu.make_async_copy(
                dy_ref.at[mine_rows, pl.ds(j * DYC, DYC)],
                dyf_ref.at[j % 2],
                dy_sems.at[j % 2],
            )

        @pl.when((c == 0) & (k == 0))
        def _():
            dy_dma(0).start()
            dy_dma(1).start()

        @pl.when(c == 0)
        def _():
            for jo in (0, 1):
                j = 2 * k + jo
                dy_dma(j).wait()

                @pl.when(j + 2 < K_SHARD // DYC)
                def _(j=j):
                    dy_dma(j + 2).start()

                dyv_ref[:, pl.ds(j * DYC, DYC)] = (
                    dyf_ref[j % 2].astype(jnp.bfloat16)
                )

        wv_ref[...] = w_ref[...].astype(jnp.bfloat16)
        part = lax.dot_general(
            dyv_ref[:, pl.ds(k * KB, KB)],
            wv_ref[...],
            (((1,), (1,)), ((), ())),
            preferred_element_type=jnp.float32,
        )

        @pl.when(k == 0)
        def _():
            acc_ref[...] = part

        @pl.when((k != 0) & (k != N_K - 1))
        def _():
            acc_ref[...] += part

        @pl.when(k == N_K - 1)
        def _():
            out_ref[mine_rows, pl.ds(c * TW, TW)] = (
                (acc_ref[...] + part).astype(jnp.bfloat16)
            )

        def xex(ch):
            cols = pl.ds(ch * CW, CW)
            return pltpu.make_async_remote_copy(
                src_ref=out_ref.at[mine_rows, cols],
                dst_ref=rx_ref.at[ch],
                send_sem=xs_sems.at[ch], recv_sem=xr_sems.at[ch],
                device_id=x_nbr, device_id_type=_MESH,
            )

        def yex(ch):
            cols = pl.ds(ch * CW, CW)
            return pltpu.make_async_remote_copy(
                src_ref=out_ref.at[mine_rows, cols],
                dst_ref=out_ref.at[mine_rows, cols],
                send_sem=ys_sems.at[ch], recv_sem=yr_sems.at[ch],
                device_id=y_nbr, device_id_type=_MESH,
            )

        def finish_x(ch):
            xex(ch).wait()
            cols = pl.ds(ch * CW, CW)
            out_ref[mine_rows, cols] = (
                out_ref[mine_rows, cols].astype(jnp.float32)
                + rx_ref[ch].astype(jnp.float32)
            ).astype(jnp.bfloat16)
            yex(ch).start()

        for cc in range(1, N_T):
            @pl.when((c == cc) & (k == N_K - 1))
            def _(cc=cc):
                finish_x(2 * (cc - 1))
                finish_x(2 * (cc - 1) + 1)

        @pl.when(k == N_K - 1)
        def _():
            xex(2 * c).start()
            xex(2 * c + 1).start()

        @pl.when((c == N_T - 1) & (k == N_K - 1))
        def _():
            finish_x(2 * (N_T - 1))
            finish_x(2 * (N_T - 1) + 1)
            for ch in range(N_C):
                yex(ch).wait()

    return pl.pallas_call(
        body,
        grid=(N_T, N_K),
        out_shape=jax.ShapeDtypeStruct((M, M), jnp.bfloat16),
        in_specs=[
            pl.BlockSpec(memory_space=pltpu.MemorySpace.HBM),
            pl.BlockSpec((TW, KB), lambda c, k: (c, k)),
        ],
        out_specs=pl.BlockSpec((M, M), lambda c, k: (0, 0)),
        scratch_shapes=[
            pltpu.VMEM((H, K_SHARD), jnp.bfloat16),
            pltpu.VMEM((2, H, DYC), jnp.float32),
            pltpu.VMEM((TW, KB), jnp.bfloat16),
            pltpu.VMEM((N_C, H, CW), jnp.bfloat16),
            pltpu.VMEM((H, TW), jnp.float32),
            pltpu.SemaphoreType.REGULAR((2,)),
            pltpu.SemaphoreType.DMA((2,)),
            pltpu.SemaphoreType.DMA((N_C,)),
            pltpu.SemaphoreType.DMA((N_C,)),
            pltpu.SemaphoreType.DMA((N_C,)),
            pltpu.SemaphoreType.DMA((N_C,)),
        ],
        compiler_params=pltpu.CompilerParams(
            vmem_limit_bytes=100 * 1024 * 1024,
        ),
    )(dy, W)
